# baseline (speedup 1.0000x reference)
import functools

import jax
import jax.numpy as jnp
from jax import lax
from jax.experimental import pallas as pl
from jax.experimental.pallas import tpu as pltpu
from jax.experimental.pallas import tpu_sc as plsc

_PERM_DNUMS = lax.GatherDimensionNumbers(
    offset_dims=(), collapsed_slice_dims=(0,), start_index_map=(0,))


def _permute(v, idx):
    return lax.gather(v, idx[:, None], _PERM_DNUMS, slice_sizes=(1,),
                      mode=lax.GatherScatterMode.PROMISE_IN_BOUNDS)


H = 128
LANES = 16
NC = 2
NS = 16
NW = NC * NS
CHUNK = 128


@functools.lru_cache(maxsize=None)
def _build(batch: int):
    assert batch % (NW * CHUNK) == 0
    kpw = batch // (NW * CHUNK)
    ppw = kpw * CHUNK
    mesh = plsc.VectorSubcoreMesh(core_axis_name="c", subcore_axis_name="s")

    @functools.partial(
        pl.kernel,
        mesh=mesh,
        out_type=jax.ShapeDtypeStruct((batch,), jnp.float32),
        scratch_types=[
            pltpu.VMEM((2 * ppw,), jnp.int32),
            pltpu.VMEM((kpw, CHUNK), jnp.int32),
            pltpu.VMEM((kpw, CHUNK), jnp.int32),
            pltpu.VMEM((2, CHUNK, H), jnp.float32),
            pltpu.VMEM((2, CHUNK, H), jnp.float32),
            pltpu.VMEM((ppw,), jnp.float32),
            pltpu.SemaphoreType.DMA,
            pltpu.SemaphoreType.DMA,
        ],
    )
    def sc_kernel(pairs_hbm, drug_hbm, cmpd_hbm, out_hbm,
                  pairs_v, idx0_v, idx1_v, drows_v, crows_v, out_v,
                  sem0, sem1):
        wid = lax.axis_index("s") * NC + lax.axis_index("c")
        sems = (sem0, sem1)
        lane = lax.broadcasted_iota(jnp.int32, (LANES,), 0)

        pltpu.sync_copy(pairs_hbm.at[pl.ds(wid * 2 * ppw, 2 * ppw)], pairs_v)

        even = jnp.bitwise_and(2 * lane, 15)
        odd = even + 1
        low_half = lane < 8

        def make_idx(j):
            for g in range(CHUNK // LANES):
                base = 2 * (j * CHUNK + g * LANES)
                v0 = pairs_v[pl.ds(base, LANES)]
                v1 = pairs_v[pl.ds(base + LANES, LANES)]
                idx0_v[j, pl.ds(g * LANES, LANES)] = jnp.where(
                    low_half, _permute(v0, even), _permute(v1, even))
                idx1_v[j, pl.ds(g * LANES, LANES)] = jnp.where(
                    low_half, _permute(v0, odd), _permute(v1, odd))

        def start_gathers(j):
            buf = j % 2
            sem = sems[buf]
            cp_d = pltpu.async_copy(
                drug_hbm.at[idx0_v.at[j]], drows_v.at[buf], sem)
            cp_c = pltpu.async_copy(
                cmpd_hbm.at[idx1_v.at[j]], crows_v.at[buf], sem)
            return cp_d, cp_c

        make_idx(0)
        pending = start_gathers(0)

        for j in range(kpw):
            if j + 1 < kpw:
                make_idx(j + 1)
                nxt = start_gathers(j + 1)
            pending[0].wait()
            pending[1].wait()
            buf = j % 2

            for g in range(CHUNK // LANES):
                def pair_body(t, vec, g=g, buf=buf):
                    b = g * LANES + t
                    acc = (drows_v[buf, b, pl.ds(0, LANES)]
                           * crows_v[buf, b, pl.ds(0, LANES)])
                    for i in range(1, H // LANES):
                        acc = acc + (drows_v[buf, b, pl.ds(i * LANES, LANES)]
                                     * crows_v[buf, b, pl.ds(i * LANES, LANES)])
                    for sh in (8, 4, 2, 1):
                        acc = acc + _permute(acc, jnp.bitwise_xor(lane, sh))
                    return jnp.where(lane == t, acc, vec)

                vec = lax.fori_loop(0, LANES, pair_body,
                                    jnp.zeros((LANES,), jnp.float32),
                                    unroll=4)
                out_v[pl.ds(j * CHUNK + g * LANES, LANES)] = vec

            if j + 1 < kpw:
                pending = nxt

        pltpu.sync_copy(out_v, out_hbm.at[pl.ds(wid * ppw, ppw)])

    return sc_kernel


def kernel(inputs, drug_table, cmpd_table):
    batch = inputs.shape[0]
    pairs = inputs.astype(jnp.int32).reshape(2 * batch)
    out = _build(batch)(pairs, drug_table, cmpd_table)
    return out.reshape(batch, 1, 1)


# speedup vs baseline: 1.1233x; 1.1233x over previous
import functools

import jax
import jax.numpy as jnp
from jax import lax
from jax.experimental import pallas as pl
from jax.experimental.pallas import tpu as pltpu
from jax.experimental.pallas import tpu_sc as plsc

_PERM_DNUMS = lax.GatherDimensionNumbers(
    offset_dims=(), collapsed_slice_dims=(0,), start_index_map=(0,))


def _permute(v, idx):
    return lax.gather(v, idx[:, None], _PERM_DNUMS, slice_sizes=(1,),
                      mode=lax.GatherScatterMode.PROMISE_IN_BOUNDS)


H = 128
LANES = 16
NC = 2
NS = 16
NW = NC * NS
CHUNK = 128


@functools.lru_cache(maxsize=None)
def _build(batch: int):
    assert batch % (NW * CHUNK) == 0
    kpw = batch // (NW * CHUNK)
    ppw = kpw * CHUNK
    mesh = plsc.VectorSubcoreMesh(core_axis_name="c", subcore_axis_name="s")

    @functools.partial(
        pl.kernel,
        mesh=mesh,
        out_type=jax.ShapeDtypeStruct((batch,), jnp.float32),
        scratch_types=[
            pltpu.VMEM((kpw, CHUNK), jnp.int32),
            pltpu.VMEM((kpw, CHUNK), jnp.int32),
            pltpu.VMEM((2, CHUNK, H), jnp.float32),
            pltpu.VMEM((2, CHUNK, H), jnp.float32),
            pltpu.VMEM((ppw,), jnp.float32),
            pltpu.SemaphoreType.DMA,
            pltpu.SemaphoreType.DMA,
        ],
    )
    def sc_kernel(idx0_hbm, idx1_hbm, drug_hbm, cmpd_hbm, out_hbm,
                  idx0_v, idx1_v, drows_v, crows_v, out_v,
                  sem0, sem1):
        wid = lax.axis_index("s") * NC + lax.axis_index("c")
        sems = (sem0, sem1)
        lane = lax.broadcasted_iota(jnp.int32, (LANES,), 0)

        pltpu.sync_copy(idx0_hbm.at[pl.ds(wid * kpw, kpw)], idx0_v)
        pltpu.sync_copy(idx1_hbm.at[pl.ds(wid * kpw, kpw)], idx1_v)

        def start_gathers(j):
            buf = j % 2
            sem = sems[buf]
            cp_d = pltpu.async_copy(
                drug_hbm.at[idx0_v.at[j]], drows_v.at[buf], sem)
            cp_c = pltpu.async_copy(
                cmpd_hbm.at[idx1_v.at[j]], crows_v.at[buf], sem)
            return cp_d, cp_c

        pending = start_gathers(0)

        for j in range(kpw):
            if j + 1 < kpw:
                nxt = start_gathers(j + 1)
            pending[0].wait()
            pending[1].wait()
            buf = j % 2

            for g in range(CHUNK // LANES):
                def pair_body(t, vec, g=g, buf=buf):
                    b = g * LANES + t
                    acc = (drows_v[buf, b, pl.ds(0, LANES)]
                           * crows_v[buf, b, pl.ds(0, LANES)])
                    for i in range(1, H // LANES):
                        acc = acc + (drows_v[buf, b, pl.ds(i * LANES, LANES)]
                                     * crows_v[buf, b, pl.ds(i * LANES, LANES)])
                    for sh in (8, 4, 2, 1):
                        acc = acc + _permute(acc, jnp.bitwise_xor(lane, sh))
                    return jnp.where(lane == t, acc, vec)

                vec = lax.fori_loop(0, LANES, pair_body,
                                    jnp.zeros((LANES,), jnp.float32),
                                    unroll=4)
                out_v[pl.ds(j * CHUNK + g * LANES, LANES)] = vec

            if j + 1 < kpw:
                pending = nxt

        pltpu.sync_copy(out_v, out_hbm.at[pl.ds(wid * ppw, ppw)])

    return sc_kernel


def kernel(inputs, drug_table, cmpd_table):
    batch = inputs.shape[0]
    idx = inputs.astype(jnp.int32)
    idx0 = idx[:, 0].reshape(batch // CHUNK, CHUNK)
    idx1 = idx[:, 1].reshape(batch // CHUNK, CHUNK)
    out = _build(batch)(idx0, idx1, drug_table, cmpd_table)
    return out.reshape(batch, 1, 1)


# speedup vs baseline: 1.1930x; 1.0620x over previous
import functools

import jax
import jax.numpy as jnp
from jax import lax
from jax.experimental import pallas as pl
from jax.experimental.pallas import tpu as pltpu
from jax.experimental.pallas import tpu_sc as plsc

_PERM_DNUMS = lax.GatherDimensionNumbers(
    offset_dims=(), collapsed_slice_dims=(0,), start_index_map=(0,))


def _permute(v, idx):
    return lax.gather(v, idx[:, None], _PERM_DNUMS, slice_sizes=(1,),
                      mode=lax.GatherScatterMode.PROMISE_IN_BOUNDS)


H = 128
LANES = 16
NC = 2
NS = 16
NW = NC * NS
CHUNK = 128


@functools.lru_cache(maxsize=None)
def _build(batch: int):
    assert batch % (NW * CHUNK) == 0
    kpw = batch // (NW * CHUNK)
    ppw = kpw * CHUNK
    mesh = plsc.VectorSubcoreMesh(core_axis_name="c", subcore_axis_name="s")

    @functools.partial(
        pl.kernel,
        mesh=mesh,
        out_type=jax.ShapeDtypeStruct((batch,), jnp.float32),
        scratch_types=[
            pltpu.VMEM((kpw, CHUNK), jnp.int32),
            pltpu.VMEM((kpw, CHUNK), jnp.int32),
            pltpu.VMEM((2, CHUNK, H), jnp.float32),
            pltpu.VMEM((2, CHUNK, H), jnp.float32),
            pltpu.VMEM((ppw,), jnp.float32),
            pltpu.SemaphoreType.DMA,
            pltpu.SemaphoreType.DMA,
        ],
    )
    def sc_kernel(idx0_hbm, idx1_hbm, drug_hbm, cmpd_hbm, out_hbm,
                  idx0_v, idx1_v, drows_v, crows_v, out_v,
                  sem0, sem1):
        wid = lax.axis_index("s") * NC + lax.axis_index("c")
        sems = (sem0, sem1)
        lane = lax.broadcasted_iota(jnp.int32, (LANES,), 0)

        pltpu.sync_copy(idx0_hbm.at[pl.ds(wid * kpw, kpw)], idx0_v)
        pltpu.sync_copy(idx1_hbm.at[pl.ds(wid * kpw, kpw)], idx1_v)

        def start_gathers(j):
            buf = j % 2
            sem = sems[buf]
            cp_d = pltpu.async_copy(
                drug_hbm.at[idx0_v.at[j]], drows_v.at[buf], sem)
            cp_c = pltpu.async_copy(
                cmpd_hbm.at[idx1_v.at[j]], crows_v.at[buf], sem)
            return cp_d, cp_c

        pending = start_gathers(0)

        for j in range(kpw):
            if j + 1 < kpw:
                nxt = start_gathers(j + 1)
            pending[0].wait()
            pending[1].wait()
            buf = j % 2

            for g in range(CHUNK // LANES):
                def pair_body(t, vec, g=g, buf=buf):
                    b = g * LANES + t
                    acc = (drows_v[buf, b, pl.ds(0, LANES)]
                           * crows_v[buf, b, pl.ds(0, LANES)])
                    for i in range(1, H // LANES):
                        acc = acc + (drows_v[buf, b, pl.ds(i * LANES, LANES)]
                                     * crows_v[buf, b, pl.ds(i * LANES, LANES)])
                    for sh in (8, 4, 2, 1):
                        acc = acc + _permute(acc, jnp.bitwise_xor(lane, sh))
                    return jnp.where(lane == t, acc, vec)

                vec = lax.fori_loop(0, LANES, pair_body,
                                    jnp.zeros((LANES,), jnp.float32),
                                    unroll=2)
                out_v[pl.ds(j * CHUNK + g * LANES, LANES)] = vec

            if j + 1 < kpw:
                pending = nxt

        pltpu.sync_copy(out_v, out_hbm.at[pl.ds(wid * ppw, ppw)])

    return sc_kernel


def kernel(inputs, drug_table, cmpd_table):
    batch = inputs.shape[0]
    idx = inputs.astype(jnp.int32)
    idx0 = idx[:, 0].reshape(batch // CHUNK, CHUNK)
    idx1 = idx[:, 1].reshape(batch // CHUNK, CHUNK)
    out = _build(batch)(idx0, idx1, drug_table, cmpd_table)
    return out.reshape(batch, 1, 1)


# speedup vs baseline: 1.2564x; 1.0532x over previous
import functools

import jax
import jax.numpy as jnp
from jax import lax
from jax.experimental import pallas as pl
from jax.experimental.pallas import tpu as pltpu
from jax.experimental.pallas import tpu_sc as plsc

_PERM_DNUMS = lax.GatherDimensionNumbers(
    offset_dims=(), collapsed_slice_dims=(0,), start_index_map=(0,))


def _permute(v, idx):
    return lax.gather(v, idx[:, None], _PERM_DNUMS, slice_sizes=(1,),
                      mode=lax.GatherScatterMode.PROMISE_IN_BOUNDS)


H = 128
LANES = 16
NC = 2
NS = 16
NW = NC * NS
CHUNK = 128


@functools.lru_cache(maxsize=None)
def _build(batch: int):
    assert batch % (NW * CHUNK) == 0
    kpw = batch // (NW * CHUNK)
    ppw = kpw * CHUNK
    mesh = plsc.VectorSubcoreMesh(core_axis_name="c", subcore_axis_name="s")

    @functools.partial(
        pl.kernel,
        mesh=mesh,
        out_type=jax.ShapeDtypeStruct((batch,), jnp.float32),
        scratch_types=[
            pltpu.VMEM((kpw, CHUNK), jnp.int32),
            pltpu.VMEM((kpw, CHUNK), jnp.int32),
            pltpu.VMEM((2, CHUNK, H), jnp.float32),
            pltpu.VMEM((2, CHUNK, H), jnp.float32),
            pltpu.VMEM((ppw,), jnp.float32),
            pltpu.SemaphoreType.DMA,
            pltpu.SemaphoreType.DMA,
        ],
    )
    def sc_kernel(idx0_hbm, idx1_hbm, drug_hbm, cmpd_hbm, out_hbm,
                  idx0_v, idx1_v, drows_v, crows_v, out_v,
                  sem0, sem1):
        wid = lax.axis_index("s") * NC + lax.axis_index("c")
        sems = (sem0, sem1)
        lane = lax.broadcasted_iota(jnp.int32, (LANES,), 0)

        pltpu.sync_copy(idx0_hbm.at[pl.ds(wid * kpw, kpw)], idx0_v)
        pltpu.sync_copy(idx1_hbm.at[pl.ds(wid * kpw, kpw)], idx1_v)

        def start_gathers(j):
            buf = j % 2
            sem = sems[buf]
            cp_d = pltpu.async_copy(
                drug_hbm.at[idx0_v.at[j]], drows_v.at[buf], sem)
            cp_c = pltpu.async_copy(
                cmpd_hbm.at[idx1_v.at[j]], crows_v.at[buf], sem)
            return cp_d, cp_c

        pending = start_gathers(0)

        for j in range(kpw):
            if j + 1 < kpw:
                nxt = start_gathers(j + 1)
            pending[0].wait()
            pending[1].wait()
            buf = j % 2

            for g in range(CHUNK // LANES):
                def pair_body(t, vec, g=g, buf=buf):
                    b = g * LANES + t
                    acc = (drows_v[buf, b, pl.ds(0, LANES)]
                           * crows_v[buf, b, pl.ds(0, LANES)])
                    for i in range(1, H // LANES):
                        acc = acc + (drows_v[buf, b, pl.ds(i * LANES, LANES)]
                                     * crows_v[buf, b, pl.ds(i * LANES, LANES)])
                    for sh in (8, 4, 2, 1):
                        acc = acc + _permute(acc, jnp.bitwise_xor(lane, sh))
                    return jnp.where(lane == t, acc, vec)

                vec = lax.fori_loop(0, LANES, pair_body,
                                    jnp.zeros((LANES,), jnp.float32))
                out_v[pl.ds(j * CHUNK + g * LANES, LANES)] = vec

            if j + 1 < kpw:
                pending = nxt

        pltpu.sync_copy(out_v, out_hbm.at[pl.ds(wid * ppw, ppw)])

    return sc_kernel


def kernel(inputs, drug_table, cmpd_table):
    batch = inputs.shape[0]
    idx = inputs.astype(jnp.int32)
    idx0 = idx[:, 0].reshape(batch // CHUNK, CHUNK)
    idx1 = idx[:, 1].reshape(batch // CHUNK, CHUNK)
    out = _build(batch)(idx0, idx1, drug_table, cmpd_table)
    return out.reshape(batch, 1, 1)


# speedup vs baseline: 1.5258x; 1.2144x over previous
import functools

import jax
import jax.numpy as jnp
from jax import lax
from jax.experimental import pallas as pl
from jax.experimental.pallas import tpu as pltpu
from jax.experimental.pallas import tpu_sc as plsc

_PERM_DNUMS = lax.GatherDimensionNumbers(
    offset_dims=(), collapsed_slice_dims=(0,), start_index_map=(0,))


def _permute(v, idx):
    return lax.gather(v, idx[:, None], _PERM_DNUMS, slice_sizes=(1,),
                      mode=lax.GatherScatterMode.PROMISE_IN_BOUNDS)


H = 128
LANES = 16
NC = 2
NS = 16
NW = NC * NS
CHUNK = 128


@functools.lru_cache(maxsize=None)
def _build(batch: int):
    assert batch % (NW * CHUNK) == 0
    kpw = batch // (NW * CHUNK)
    ppw = kpw * CHUNK
    mesh = plsc.VectorSubcoreMesh(core_axis_name="c", subcore_axis_name="s")

    @functools.partial(
        pl.kernel,
        mesh=mesh,
        out_type=jax.ShapeDtypeStruct((batch,), jnp.float32),
        scratch_types=[
            pltpu.VMEM((kpw, CHUNK), jnp.int32),
            pltpu.VMEM((kpw, CHUNK), jnp.int32),
            pltpu.VMEM((2, CHUNK, H), jnp.float32),
            pltpu.VMEM((2, CHUNK, H), jnp.float32),
            pltpu.VMEM((ppw,), jnp.float32),
            pltpu.SemaphoreType.DMA,
            pltpu.SemaphoreType.DMA,
        ],
    )
    def sc_kernel(idx0_hbm, idx1_hbm, drug_hbm, cmpd_hbm, out_hbm,
                  idx0_v, idx1_v, drows_v, crows_v, out_v,
                  sem0, sem1):
        wid = lax.axis_index("s") * NC + lax.axis_index("c")
        sems = (sem0, sem1)
        lane = lax.broadcasted_iota(jnp.int32, (LANES,), 0)

        pltpu.sync_copy(idx0_hbm.at[pl.ds(wid * kpw, kpw)], idx0_v)
        pltpu.sync_copy(idx1_hbm.at[pl.ds(wid * kpw, kpw)], idx1_v)

        def start_gathers(j):
            buf = j % 2
            sem = sems[buf]
            cp_d = pltpu.async_copy(
                drug_hbm.at[idx0_v.at[j]], drows_v.at[buf], sem)
            cp_c = pltpu.async_copy(
                cmpd_hbm.at[idx1_v.at[j]], crows_v.at[buf], sem)
            return cp_d, cp_c

        pending = start_gathers(0)

        for j in range(kpw):
            if j + 1 < kpw:
                nxt = start_gathers(j + 1)
            pending[0].wait()
            pending[1].wait()
            buf = j % 2

            def group_body(g, carry, j=j, buf=buf):
                def pair_body(t, vec):
                    b = g * LANES + t
                    acc = (drows_v[buf, b, pl.ds(0, LANES)]
                           * crows_v[buf, b, pl.ds(0, LANES)])
                    for i in range(1, H // LANES):
                        acc = acc + (drows_v[buf, b, pl.ds(i * LANES, LANES)]
                                     * crows_v[buf, b, pl.ds(i * LANES, LANES)])
                    for sh in (8, 4, 2, 1):
                        acc = acc + _permute(acc, jnp.bitwise_xor(lane, sh))
                    return jnp.where(lane == t, acc, vec)

                vec = lax.fori_loop(0, LANES, pair_body,
                                    jnp.zeros((LANES,), jnp.float32))
                out_v[pl.ds(j * CHUNK + g * LANES, LANES)] = vec
                return carry

            lax.fori_loop(0, CHUNK // LANES, group_body, 0)

            if j + 1 < kpw:
                pending = nxt

        pltpu.sync_copy(out_v, out_hbm.at[pl.ds(wid * ppw, ppw)])

    return sc_kernel


def kernel(inputs, drug_table, cmpd_table):
    batch = inputs.shape[0]
    idx = inputs.astype(jnp.int32)
    idx0 = idx[:, 0].reshape(batch // CHUNK, CHUNK)
    idx1 = idx[:, 1].reshape(batch // CHUNK, CHUNK)
    out = _build(batch)(idx0, idx1, drug_table, cmpd_table)
    return out.reshape(batch, 1, 1)
